# baseline (speedup 1.0000x reference)
import math

import jax
import jax.numpy as jnp
from jax.experimental import pallas as pl
from jax.experimental.pallas import tpu as pltpu

B = 16384
D = 1024
HALF = D // 2
R = 512
GRID = B // R


def _tc_body(pos_ref, div_ref, out_ref):
    pos = pos_ref[...]
    div = div_ref[...]
    x = pos * div
    out_ref[:, :HALF] = jnp.sin(x)
    out_ref[:, HALF:] = jnp.cos(x)


def kernel(positions, pe):
    flat = positions.reshape(-1).astype(jnp.float32).reshape(B, 1)
    half = HALF
    scale = -math.log(10000.0) / (half - 1)
    div_term = jnp.exp(jnp.arange(half, dtype=jnp.float32) * scale)
    div_term = div_term.reshape(1, half)

    out = pl.pallas_call(
        _tc_body,
        out_shape=jax.ShapeDtypeStruct((B, D), jnp.float32),
        grid=(GRID,),
        in_specs=[
            pl.BlockSpec((R, 1), lambda i: (i, 0)),
            pl.BlockSpec((1, half), lambda i: (0, 0)),
        ],
        out_specs=pl.BlockSpec((R, D), lambda i: (i, 0)),
    )(flat, div_term)
    return out.reshape(*positions.shape, pe.shape[-1])


# speedup vs baseline: 1.1812x; 1.1812x over previous
import math

import jax
import jax.numpy as jnp
from jax import lax
from jax.experimental import pallas as pl
from jax.experimental.pallas import tpu as pltpu
from jax.experimental.pallas import tpu_sc as plsc

NC = 2
NS = 16
NW = NC * NS

B = 16384
D = 1024
HALF = D // 2

B_SC = 10240
B_TC = B - B_SC
SC_PER_W = B_SC // NW
CHUNK = 32
NBUF = 3
NCHUNK = SC_PER_W // CHUNK

R_TC = 512
TC_GRID = B_TC // R_TC


def _sc_gather(idx_hbm, table_hbm, out_hbm, idx_v, rows_v, in_sem, out_sem):
    wid = lax.axis_index("s") * NC + lax.axis_index("c")
    base = wid * SC_PER_W
    pltpu.sync_copy(idx_hbm.at[pl.ds(base, SC_PER_W)], idx_v)

    def gather(g, buf):
        return pltpu.make_async_copy(
            table_hbm.at[idx_v.at[pl.ds(g * CHUNK, CHUNK)]],
            rows_v.at[buf],
            in_sem,
        )

    def put(g, buf):
        return pltpu.make_async_copy(
            rows_v.at[buf],
            out_hbm.at[pl.ds(base + g * CHUNK, CHUNK)],
            out_sem,
        )

    gather(0, 0).start()
    for g in range(NCHUNK):
        buf = g % NBUF
        nxt = (g + 1) % NBUF
        if g + 1 < NCHUNK:
            if g + 1 >= NBUF:
                put(g + 1 - NBUF, nxt).wait()
            gather(g + 1, nxt).start()
        gather(g, buf).wait()
        put(g, buf).start()
    for g in range(NCHUNK - NBUF, NCHUNK):
        put(g, g % NBUF).wait()


def _tc_body(pos_ref, div_ref, out_ref):
    pos = pos_ref[...]
    div = div_ref[...]
    x = pos * div
    out_ref[:, :HALF] = jnp.sin(x)
    out_ref[:, HALF:] = jnp.cos(x)


def kernel(positions, pe):
    flat = positions.reshape(-1)

    mesh = plsc.VectorSubcoreMesh(core_axis_name="c", subcore_axis_name="s")
    sc_out = pl.kernel(
        _sc_gather,
        out_type=jax.ShapeDtypeStruct((B_SC, D), jnp.float32),
        mesh=mesh,
        scratch_types=[
            pltpu.VMEM((SC_PER_W,), jnp.int32),
            pltpu.VMEM((NBUF, CHUNK, D), jnp.float32),
            pltpu.SemaphoreType.DMA,
            pltpu.SemaphoreType.DMA,
        ],
    )(flat[:B_SC], pe)

    pos_tc = flat[B_SC:].astype(jnp.float32).reshape(B_TC, 1)
    scale = -math.log(10000.0) / (HALF - 1)
    div_term = jnp.exp(jnp.arange(HALF, dtype=jnp.float32) * scale)
    div_term = div_term.reshape(1, HALF)

    tc_out = pl.pallas_call(
        _tc_body,
        out_shape=jax.ShapeDtypeStruct((B_TC, D), jnp.float32),
        grid=(TC_GRID,),
        in_specs=[
            pl.BlockSpec((R_TC, 1), lambda i: (i, 0)),
            pl.BlockSpec((1, HALF), lambda i: (0, 0)),
        ],
        out_specs=pl.BlockSpec((R_TC, D), lambda i: (i, 0)),
    )(pos_tc, div_term)

    out = jnp.concatenate([sc_out, tc_out], axis=0)
    return out.reshape(*positions.shape, pe.shape[-1])


# speedup vs baseline: 1.5299x; 1.2952x over previous
import math

import jax
import jax.numpy as jnp
from jax import lax
from jax.experimental import pallas as pl
from jax.experimental.pallas import tpu as pltpu
from jax.experimental.pallas import tpu_sc as plsc

NC = 2
NS = 16
NW = NC * NS

B = 16384
D = 1024
HALF = D // 2

B_SC = 10240
B_TC = B - B_SC
SC_PER_W = B_SC // NW
CHUNK = 32
NBUF = 3
NCHUNK = SC_PER_W // CHUNK

R_TC = 512
TC_GRID = B_TC // R_TC


def _sc_gather(idx_hbm, table_hbm, out_hbm, idx_v, rows_v, in_sem, out_sem):
    wid = lax.axis_index("s") * NC + lax.axis_index("c")
    base = wid * SC_PER_W
    pltpu.sync_copy(idx_hbm.at[pl.ds(base, SC_PER_W)], idx_v)

    def gather(g, buf):
        return pltpu.make_async_copy(
            table_hbm.at[idx_v.at[pl.ds(g * CHUNK, CHUNK)]],
            rows_v.at[buf],
            in_sem,
        )

    def put(g, buf):
        return pltpu.make_async_copy(
            rows_v.at[buf],
            out_hbm.at[pl.ds(base + g * CHUNK, CHUNK)],
            out_sem,
        )

    gather(0, 0).start()
    for g in range(NCHUNK):
        buf = g % NBUF
        nxt = (g + 1) % NBUF
        if g + 1 < NCHUNK:
            if g + 1 >= NBUF:
                put(g + 1 - NBUF, nxt).wait()
            gather(g + 1, nxt).start()
        gather(g, buf).wait()
        put(g, buf).start()
    for g in range(NCHUNK - NBUF, NCHUNK):
        put(g, g % NBUF).wait()


def _tc_body(pos_ref, div_ref, out_ref):
    pos = pos_ref[...]
    div = div_ref[...]
    x = pos * div
    out_ref[:, :HALF] = jnp.sin(x)
    out_ref[:, HALF:] = jnp.cos(x)


def kernel(positions, pe):
    flat = positions.reshape(-1)

    mesh = plsc.VectorSubcoreMesh(core_axis_name="c", subcore_axis_name="s")
    sc_out = pl.kernel(
        _sc_gather,
        out_type=jax.ShapeDtypeStruct((B, D), jnp.float32),
        mesh=mesh,
        scratch_types=[
            pltpu.VMEM((SC_PER_W,), jnp.int32),
            pltpu.VMEM((NBUF, CHUNK, D), jnp.float32),
            pltpu.SemaphoreType.DMA,
            pltpu.SemaphoreType.DMA,
        ],
    )(flat[:B_SC], pe)

    pos_tc = flat[B_SC:].astype(jnp.float32).reshape(B_TC, 1)
    scale = -math.log(10000.0) / (HALF - 1)
    div_term = jnp.exp(jnp.arange(HALF, dtype=jnp.float32) * scale)
    div_term = div_term.reshape(1, HALF)

    tc_out = pl.pallas_call(
        _tc_body,
        out_shape=jax.ShapeDtypeStruct((B_TC, D), jnp.float32),
        grid=(TC_GRID,),
        in_specs=[
            pl.BlockSpec((R_TC, 1), lambda i: (i, 0)),
            pl.BlockSpec((1, HALF), lambda i: (0, 0)),
        ],
        out_specs=pl.BlockSpec((R_TC, D), lambda i: (i, 0)),
    )(pos_tc, div_term)

    out = lax.dynamic_update_slice(sc_out, tc_out, (B_SC, 0))
    return out.reshape(*positions.shape, pe.shape[-1])


# speedup vs baseline: 1.9467x; 1.2724x over previous
import jax
import jax.numpy as jnp
from jax import lax
from jax.experimental import pallas as pl
from jax.experimental.pallas import tpu as pltpu
from jax.experimental.pallas import tpu_sc as plsc

NC = 2
NS = 16
NW = NC * NS

B = 16384
D = 1024
B_PER_W = B // NW
CHUNK = 32
NBUF = 3
NCHUNK = B_PER_W // CHUNK


def _gather_kernel(idx_hbm, table_hbm, out_hbm, idx_v, rows_v, in_sem, out_sem,
                   idx_sem):
    wid = lax.axis_index("s") * NC + lax.axis_index("c")
    base = wid * B_PER_W
    head = 2 * CHUNK
    pltpu.sync_copy(idx_hbm.at[pl.ds(base, head)], idx_v.at[pl.ds(0, head)])
    idx_rest = pltpu.make_async_copy(
        idx_hbm.at[pl.ds(base + head, B_PER_W - head)],
        idx_v.at[pl.ds(head, B_PER_W - head)],
        idx_sem,
    )
    idx_rest.start()

    def gather(g, buf):
        return pltpu.make_async_copy(
            table_hbm.at[idx_v.at[pl.ds(g * CHUNK, CHUNK)]],
            rows_v.at[buf],
            in_sem,
        )

    def put(g, buf):
        return pltpu.make_async_copy(
            rows_v.at[buf],
            out_hbm.at[pl.ds(base + g * CHUNK, CHUNK)],
            out_sem,
        )

    gather(0, 0).start()
    for g in range(NCHUNK):
        buf = g % NBUF
        nxt = (g + 1) % NBUF
        if g + 1 < NCHUNK:
            if g + 1 >= NBUF:
                put(g + 1 - NBUF, nxt).wait()
            if g == 1:
                idx_rest.wait()
            gather(g + 1, nxt).start()
        gather(g, buf).wait()
        put(g, buf).start()
    for g in range(NCHUNK - NBUF, NCHUNK):
        put(g, g % NBUF).wait()


def kernel(positions, pe):
    flat = positions.reshape(-1)
    mesh = plsc.VectorSubcoreMesh(core_axis_name="c", subcore_axis_name="s")
    out = pl.kernel(
        _gather_kernel,
        out_type=jax.ShapeDtypeStruct((B, D), jnp.float32),
        mesh=mesh,
        scratch_types=[
            pltpu.VMEM((B_PER_W,), jnp.int32),
            pltpu.VMEM((NBUF, CHUNK, D), jnp.float32),
            pltpu.SemaphoreType.DMA,
            pltpu.SemaphoreType.DMA,
            pltpu.SemaphoreType.DMA,
        ],
    )(flat, pe)
    return out.reshape(*positions.shape, pe.shape[-1])


# speedup vs baseline: 1.9495x; 1.0015x over previous
import jax
import jax.numpy as jnp
from jax import lax
from jax.experimental import pallas as pl
from jax.experimental.pallas import tpu as pltpu
from jax.experimental.pallas import tpu_sc as plsc

NC = 2
NS = 16
NW = NC * NS

B = 16384
COLS = 4096
D = 1024
B_PER_W = B // NW
CHUNK = 32
NBUF = 3
NCHUNK = B_PER_W // CHUNK


def _gather_kernel(idx_hbm, table_hbm, out_hbm, idx_v, rows_v, in_sem, out_sem,
                   idx_sem):
    wid = lax.axis_index("s") * NC + lax.axis_index("c")
    base = wid * B_PER_W
    head = 2 * CHUNK
    pltpu.sync_copy(idx_hbm.at[pl.ds(base, head)], idx_v.at[pl.ds(0, head)])
    idx_rest = pltpu.make_async_copy(
        idx_hbm.at[pl.ds(base + head, B_PER_W - head)],
        idx_v.at[pl.ds(head, B_PER_W - head)],
        idx_sem,
    )
    idx_rest.start()

    def gather(g, buf):
        return pltpu.make_async_copy(
            table_hbm.at[idx_v.at[pl.ds(g * CHUNK, CHUNK)]],
            rows_v.at[buf],
            in_sem,
        )

    def put(g, buf):
        return pltpu.make_async_copy(
            rows_v.at[buf],
            out_hbm.at[pl.ds(base + g * CHUNK, CHUNK)],
            out_sem,
        )

    gather(0, 0).start()
    for g in range(NCHUNK):
        buf = g % NBUF
        nxt = (g + 1) % NBUF
        if g + 1 < NCHUNK:
            if g + 1 >= NBUF:
                put(g + 1 - NBUF, nxt).wait()
            if g == 1:
                idx_rest.wait()
            gather(g + 1, nxt).start()
        gather(g, buf).wait()
        put(g, buf).start()
    for g in range(NCHUNK - NBUF, NCHUNK):
        put(g, g % NBUF).wait()


def kernel(positions, pe):
    flat = positions.reshape(-1)
    mesh = plsc.VectorSubcoreMesh(core_axis_name="c", subcore_axis_name="s")
    out = pl.kernel(
        _gather_kernel,
        out_type=jax.ShapeDtypeStruct((B, D), jnp.float32),
        mesh=mesh,
        scratch_types=[
            pltpu.VMEM((B_PER_W,), jnp.int32),
            pltpu.VMEM((NBUF, CHUNK, D), jnp.float32),
            pltpu.SemaphoreType.DMA,
            pltpu.SemaphoreType.DMA,
            pltpu.SemaphoreType.DMA,
        ],
    )(flat, pe)
    return out.reshape(*positions.shape, pe.shape[-1])
